# Initial kernel scaffold; baseline (speedup 1.0000x reference)
import jax
import jax.numpy as jnp
from jax.experimental import pallas as pl


def kernel(x, edge_index, edge_vals, W, b):
    raise NotImplementedError("write your pallas kernel here")


# speedup vs baseline: 6.7170x; 6.7170x over previous
import functools

import jax
import jax.numpy as jnp
from jax import lax
from jax.experimental import pallas as pl
from jax.experimental.pallas import tpu as pltpu
from jax.experimental.pallas import tpu_sc as plsc

NC = 2
NS = 16
LANES = 16
CHUNK = 80


def _sc_aggregate(x, pkdr, valr, n_chunks, n_nodes, d):
    rows0 = (n_nodes // NS) // 8 * 8
    rows_last = n_nodes - (NS - 1) * rows0
    assert rows_last % 8 == 0 and rows_last >= rows0
    zr = 16
    assert rows0 % zr == 0 and rows_last % zr == 0
    d_slices = d // LANES
    groups = CHUNK // LANES

    mesh = plsc.VectorSubcoreMesh(core_axis_name="c", subcore_axis_name="s")

    @functools.partial(
        pl.kernel,
        mesh=mesh,
        out_type=jax.ShapeDtypeStruct((NC, n_nodes, d), jnp.float32),
        scratch_types=[
            pltpu.VMEM((n_chunks, CHUNK), jnp.int32),
            pltpu.VMEM((n_chunks, CHUNK), jnp.float32),
            pltpu.VMEM((CHUNK,), jnp.int32),
            pltpu.VMEM((CHUNK,), jnp.int32),
            pltpu.VMEM((CHUNK, d), jnp.float32),
            pltpu.VMEM((zr, d), jnp.float32),
            pltpu.VMEM_SHARED((n_nodes, d), jnp.float32),
            pltpu.SemaphoreType.DMA,
        ],
    )
    def body(x_hbm, pkd_hbm, val_hbm, out_hbm,
             pkd_v, val_v, src_c, dst_c, rows_v, zbuf_v, acc_sh, sem):
        cid = lax.axis_index("c")
        sid = lax.axis_index("s")

        pltpu.sync_copy(pkd_hbm.at[cid, sid], pkd_v)
        pltpu.sync_copy(val_hbm.at[cid, sid], val_v)

        for r in range(zr):
            for j in range(d_slices):
                zbuf_v[r, pl.ds(j * LANES, LANES)] = jnp.zeros(
                    (LANES,), jnp.float32)
        base = pl.multiple_of(sid * rows0, 8)
        for t in range(rows0 // zr):
            pltpu.sync_copy(zbuf_v, acc_sh.at[pl.ds(base + t * zr, zr)])

        @pl.when(sid == NS - 1)
        def _zero_tail():
            for t in range(rows0 // zr, rows_last // zr):
                pltpu.sync_copy(zbuf_v, acc_sh.at[pl.ds(base + t * zr, zr)])
        plsc.subcore_barrier()

        def chunk_body(k, _):
            def unpack(g, _):
                sl = pl.ds(g * LANES, LANES)
                pkd16 = pkd_v[k, sl]
                src_c[sl] = lax.shift_right_logical(pkd16, 16)
                dst_c[sl] = lax.bitwise_and(pkd16, 0xFFFF)
                return 0
            lax.fori_loop(0, groups, unpack, 0)

            pltpu.async_copy(x_hbm.at[src_c], rows_v, sem).wait()

            def scale(g, _):
                vals16 = val_v[k, pl.ds(g * LANES, LANES)]
                for l in range(LANES):
                    v = vals16[l]
                    e = g * LANES + l
                    for j in range(d_slices):
                        sl = pl.ds(j * LANES, LANES)
                        rows_v[e, sl] = rows_v[e, sl] * v
                return 0
            lax.fori_loop(0, groups, scale, 0)

            pltpu.sync_copy(rows_v, acc_sh.at[dst_c], add=True)
            return 0
        lax.fori_loop(0, n_chunks, chunk_body, 0)

        plsc.subcore_barrier()

        @pl.when(sid < NS - 1)
        def _wb_main():
            sl = pl.ds(base, rows0)
            pltpu.sync_copy(acc_sh.at[sl], out_hbm.at[cid, sl])

        @pl.when(sid == NS - 1)
        def _wb_tail():
            sl = pl.ds(base, rows_last)
            pltpu.sync_copy(acc_sh.at[sl], out_hbm.at[cid, sl])

    return body(x, pkdr, valr)


def _tc_finish(partials, W, b, n_nodes, d_in, d_out, block_m):
    def body(p_ref, w_ref, b_ref, o_ref):
        a = p_ref[0] + p_ref[1]
        acc = jnp.dot(a, w_ref[...], preferred_element_type=jnp.float32)
        o_ref[...] = jnp.maximum(acc + b_ref[...], 0.0)

    return pl.pallas_call(
        body,
        grid=(n_nodes // block_m,),
        in_specs=[
            pl.BlockSpec((NC, block_m, d_in), lambda i: (0, i, 0)),
            pl.BlockSpec((d_in, d_out), lambda i: (0, 0)),
            pl.BlockSpec((1, d_out), lambda i: (0, 0)),
        ],
        out_specs=pl.BlockSpec((block_m, d_out), lambda i: (i, 0)),
        out_shape=jax.ShapeDtypeStruct((n_nodes, d_out), jnp.float32),
    )(partials, W, b.reshape(1, d_out))


def kernel(x, edge_index, edge_vals, W, b):
    n_nodes, d_in = x.shape
    d_out = W.shape[1]
    e = edge_vals.shape[0]
    workers = NC * NS
    per_worker = e // workers
    n_chunks = per_worker // CHUNK
    assert per_worker * workers == e and n_chunks * CHUNK == per_worker
    assert n_nodes < (1 << 16)

    ei = edge_index.astype(jnp.int32)
    pkd = jnp.bitwise_or(jnp.left_shift(ei[0], 16), ei[1])
    pkdr = pkd.reshape(NC, NS, n_chunks, CHUNK)
    valr = edge_vals.reshape(NC, NS, n_chunks, CHUNK)

    partials = _sc_aggregate(x, pkdr, valr, n_chunks, n_nodes, d_in)
    return _tc_finish(partials, W, b, n_nodes, d_in, d_out, block_m=1000)
